# Initial kernel scaffold; baseline (speedup 1.0000x reference)
import jax
import jax.numpy as jnp
from jax.experimental import pallas as pl


def kernel(adj, x_enc, W1, b1, W2, b2, Wm, bm, Ww, bw, Wd, bd):
    raise NotImplementedError("write your pallas kernel here")


# speedup vs baseline: 1.2911x; 1.2911x over previous
import jax
import jax.numpy as jnp
from jax.experimental import pallas as pl
from jax.experimental.pallas import tpu as pltpu

B, T, N = 16, 96, 1024
D_MODEL = 512
BT = B * T


def _dp_kernel(adjT_ref, z_ref, w1t_ref, b1_ref, w2t_ref, b2_ref,
               wcat_ref, bcat_ref, out_ref, ab_ref, ya_ref, y2_ref):
    f32 = jnp.float32
    bf16 = jnp.bfloat16

    adjT = adjT_ref[...]
    r = jax.lax.broadcasted_iota(jnp.int32, (N, N), 0)
    c = jax.lax.broadcasted_iota(jnp.int32, (N, N), 1)
    at = jnp.where(r == c, 0.0, adjT)
    deg = jnp.maximum(jnp.sum(at, axis=0, keepdims=True), 1e-6)
    ab_ref[:, :N] = (at / deg).astype(bf16)
    cmax = jnp.max(at, axis=0, keepdims=True)
    e = jnp.exp(at - cmax)
    ab_ref[:, N:] = (e / jnp.sum(e, axis=0, keepdims=True)).astype(bf16)

    z = z_ref[...]
    known = (jnp.sum(z, axis=0, keepdims=True) != 0.0).astype(f32)
    zk = (z * known).astype(bf16)

    ab = ab_ref[...]
    ya_ref[...] = jax.lax.dot_general(
        zk, ab, (((1,), (0,)), ((), ())),
        preferred_element_type=f32).astype(bf16)
    y2_ref[...] = jax.lax.dot_general(
        ya_ref[:, :N], ab[:, :N], (((1,), (0,)), ((), ())),
        preferred_element_type=f32).astype(bf16)

    w1t = w1t_ref[...].astype(bf16)
    w2t = w2t_ref[...].astype(bf16)
    b1 = b1_ref[...]
    b2 = b2_ref[...]
    wcat = wcat_ref[...].astype(bf16)
    bm = bcat_ref[:, 0:1]
    bw = bcat_ref[:, 1:2]
    bd = bcat_ref[:, 2:3]

    def body(b, carry):
        xb = z_ref[pl.ds(b * T, T), :]
        h = jax.lax.dot_general(w1t, xb.astype(bf16), (((1,), (0,)), ((), ())),
                                preferred_element_type=f32) + b1
        h = jnp.maximum(h, 0.0).astype(bf16)
        lg = jax.lax.dot_general(w2t, h, (((1,), (0,)), ((), ())),
                                 preferred_element_type=f32) + b2
        l0, l1, l2 = lg[0:1], lg[1:2], lg[2:3]
        lmax = jnp.maximum(jnp.maximum(l0, l1), l2)
        lmin = jnp.minimum(jnp.minimum(l0, l1), l2)
        is0, is1, is2 = (l0 == lmin), (l1 == lmin), (l2 == lmin)
        ex2 = is2
        ex1 = is1 & ~is2
        ex0 = is0 & ~(is1 | is2)
        w0 = jnp.where(ex0, 0.0, jnp.exp(l0 - lmax))
        w1 = jnp.where(ex1, 0.0, jnp.exp(l1 - lmax))
        w2 = jnp.where(ex2, 0.0, jnp.exp(l2 - lmax))
        inv = 1.0 / (w0 + w1 + w2)
        w0, w1, w2 = w0 * inv, w1 * inv, w2 * inv

        y1b = ya_ref[pl.ds(b * T, T), 0:N]
        ywmb = ya_ref[pl.ds(b * T, T), N:]
        y2b = y2_ref[pl.ds(b * T, T), :]
        s = jnp.concatenate([
            (y1b * w0).astype(bf16),
            (ywmb * w1).astype(bf16),
            (y2b * w2).astype(bf16),
        ], axis=0)
        combined = jax.lax.dot_general(wcat, s, (((1,), (0,)), ((), ())),
                                       preferred_element_type=f32)
        combined = combined + bm * w0 + bw * w1 + bd * w2
        out_ref[pl.ds(b * T, T), :] = jnp.where(xb == 0.0, combined, xb)
        return carry

    jax.lax.fori_loop(0, B, body, 0)


def kernel(adj, x_enc, W1, b1, W2, b2, Wm, bm, Ww, bw, Wd, bd):
    z = x_enc.reshape(BT, N)
    adjT = jnp.swapaxes(adj, 0, 1)
    w1t = jnp.swapaxes(W1, 0, 1)
    w2t = jnp.swapaxes(W2, 0, 1)
    wcat = jnp.swapaxes(jnp.concatenate([Wm, Ww, Wd], axis=0), 0, 1)
    bcat = jnp.stack([bm, bw, bd], axis=1)
    out = pl.pallas_call(
        _dp_kernel,
        out_shape=jax.ShapeDtypeStruct((BT, N), jnp.float32),
        scratch_shapes=[
            pltpu.VMEM((N, 2 * N), jnp.bfloat16),
            pltpu.VMEM((BT, 2 * N), jnp.bfloat16),
            pltpu.VMEM((BT, N), jnp.bfloat16),
        ],
    )(adjT, z, w1t, b1[:, None], w2t, b2[:, None], wcat, bcat)
    return out.reshape(B, T, N, 1)
